# Initial kernel scaffold; baseline (speedup 1.0000x reference)
import jax
import jax.numpy as jnp
from jax.experimental import pallas as pl


def kernel(idx, emb1, W, b):
    raise NotImplementedError("write your pallas kernel here")


# speedup vs baseline: 7.5605x; 7.5605x over previous
import functools

import jax
import jax.numpy as jnp
from jax.experimental import pallas as pl
from jax.experimental.pallas import tpu as pltpu

_ALPHA = 3.0
_K = 32
_RBLK = 128
_ONE_BITS = 0x3F800000


def _nv_kernel(emb_ref, wt_ref, b_ref, out_ref):
    y = jnp.dot(emb_ref[...], wt_ref[...], preferred_element_type=jnp.float32)
    out_ref[...] = jnp.tanh(_ALPHA * (y + b_ref[...]))


def _adj_kernel(nv_ref, nvt_ref, out_ref, *, n_cols):
    a = jnp.dot(nv_ref[...], nvt_ref[...], preferred_element_type=jnp.float32)
    v = jnp.maximum(jnp.tanh(_ALPHA * a), 0.0)
    col = jax.lax.broadcasted_iota(jnp.int32, v.shape, 1)
    v = jnp.where(col < n_cols, v, 0.0)
    u = jax.lax.bitcast_convert_type(v, jnp.int32)

    rows = v.shape[0]

    def body(_, carry):
        lo, hi = carry
        mid = (lo + hi + 1) >> 1
        cnt = jnp.sum((u >= mid).astype(jnp.int32), axis=1, keepdims=True)
        ok = cnt >= _K
        return jnp.where(ok, mid, lo), jnp.where(ok, hi, mid)

    lo0 = jnp.zeros((rows, 1), jnp.int32)
    hi0 = jnp.full((rows, 1), _ONE_BITS + 1, jnp.int32)
    tau, _ = jax.lax.fori_loop(0, 31, body, (lo0, hi0))

    gt = u > tau
    cnt_gt = jnp.sum(gt.astype(jnp.int32), axis=1, keepdims=True)
    need = _K - cnt_gt
    tie = u == tau

    def ibody(_, carry):
        lo, hi = carry
        mid = (lo + hi) >> 1
        cnt = jnp.sum((tie & (col <= mid)).astype(jnp.int32), axis=1,
                      keepdims=True)
        ok = cnt >= need
        return jnp.where(ok, lo, mid), jnp.where(ok, mid, hi)

    ncols_pad = v.shape[1]
    ilo0 = jnp.full((rows, 1), -1, jnp.int32)
    ihi0 = jnp.full((rows, 1), ncols_pad - 1, jnp.int32)
    nbits = max(1, (ncols_pad - 1).bit_length())
    _, iot = jax.lax.fori_loop(0, nbits, ibody, (ilo0, ihi0))

    keep = gt | (tie & (col <= iot))
    res = jnp.where(keep, v, 0.0)
    out_ref[...] = res[:, :n_cols]


def kernel(idx, emb1, W, b):
    n, d = emb1.shape
    x = jnp.take(emb1, idx, axis=0)
    npad = ((n + _RBLK - 1) // _RBLK) * _RBLK
    xp = jnp.pad(x, ((0, npad - n), (0, 0)))
    wt = W.T
    b2 = b.reshape(1, d)

    nv = pl.pallas_call(
        _nv_kernel,
        out_shape=jax.ShapeDtypeStruct((npad, d), jnp.float32),
    )(xp, wt, b2)
    nvt = nv.T

    grid = npad // _RBLK
    adj = pl.pallas_call(
        functools.partial(_adj_kernel, n_cols=n),
        grid=(grid,),
        in_specs=[
            pl.BlockSpec((_RBLK, d), lambda i: (i, 0)),
            pl.BlockSpec((d, npad), lambda i: (0, 0)),
        ],
        out_specs=pl.BlockSpec((_RBLK, n), lambda i: (i, 0)),
        out_shape=jax.ShapeDtypeStruct((n, n), jnp.float32),
        compiler_params=pltpu.CompilerParams(
            dimension_semantics=("arbitrary",)
        ),
    )(nv, nvt)
    return adj


# speedup vs baseline: 16.9957x; 2.2480x over previous
import functools

import jax
import jax.numpy as jnp
from jax.experimental import pallas as pl
from jax.experimental.pallas import tpu as pltpu

_ALPHA = 3.0
_K = 32
_RBLK = 128
_ONE_BITS = 0x3F800000


def _nv_kernel(emb_ref, wt_ref, b_ref, out_ref):
    y = jnp.dot(emb_ref[...], wt_ref[...], preferred_element_type=jnp.float32)
    out_ref[...] = jnp.tanh(_ALPHA * (y + b_ref[...]))


def _adj_kernel(nv_ref, nvt_ref, out_ref, *, n_cols):
    a = jnp.dot(nv_ref[...], nvt_ref[...], preferred_element_type=jnp.float32)
    v = jnp.maximum(jnp.tanh(_ALPHA * a), 0.0)
    col = jax.lax.broadcasted_iota(jnp.int32, v.shape, 1)
    v = jnp.where(col < n_cols, v, 0.0)
    u = jax.lax.bitcast_convert_type(v, jnp.int32)

    rows = v.shape[0]

    def body(_, carry):
        lo, hi = carry
        mid = (lo + hi + 1) >> 1
        cnt = jnp.sum((u >= mid).astype(jnp.int32), axis=1, keepdims=True)
        ok = cnt >= _K
        return jnp.where(ok, mid, lo), jnp.where(ok, hi, mid)

    def _full_search():
        lo0 = jnp.zeros((rows, 1), jnp.int32)
        hi0 = jnp.full((rows, 1), _ONE_BITS + 1, jnp.int32)
        tau_s, _ = jax.lax.fori_loop(0, 31, body, (lo0, hi0))
        return tau_s

    c1 = jnp.sum((u >= _ONE_BITS).astype(jnp.int32), axis=1, keepdims=True)
    all_sat = jnp.min(c1) >= _K
    tau = jax.lax.cond(
        all_sat, lambda: jnp.full((rows, 1), _ONE_BITS, jnp.int32),
        _full_search)

    gt = u > tau
    cnt_gt = jnp.sum(gt.astype(jnp.int32), axis=1, keepdims=True)
    need = _K - cnt_gt
    tie = u == tau

    def ibody(_, carry):
        lo, hi = carry
        mid = (lo + hi) >> 1
        cnt = jnp.sum((tie & (col <= mid)).astype(jnp.int32), axis=1,
                      keepdims=True)
        ok = cnt >= need
        return jnp.where(ok, lo, mid), jnp.where(ok, mid, hi)

    ncols_pad = v.shape[1]
    ilo0 = jnp.full((rows, 1), -1, jnp.int32)
    ihi0 = jnp.full((rows, 1), ncols_pad - 1, jnp.int32)
    nbits = max(1, (ncols_pad - 1).bit_length())
    _, iot = jax.lax.fori_loop(0, nbits, ibody, (ilo0, ihi0))

    keep = gt | (tie & (col <= iot))
    res = jnp.where(keep, v, 0.0)
    out_ref[...] = res[:, :n_cols]


def kernel(idx, emb1, W, b):
    n, d = emb1.shape
    x = jnp.take(emb1, idx, axis=0)
    npad = ((n + _RBLK - 1) // _RBLK) * _RBLK
    xp = jnp.pad(x, ((0, npad - n), (0, 0)))
    wt = W.T
    b2 = b.reshape(1, d)

    nv = pl.pallas_call(
        _nv_kernel,
        out_shape=jax.ShapeDtypeStruct((npad, d), jnp.float32),
    )(xp, wt, b2)
    nvt = nv.T

    grid = npad // _RBLK
    adj = pl.pallas_call(
        functools.partial(_adj_kernel, n_cols=n),
        grid=(grid,),
        in_specs=[
            pl.BlockSpec((_RBLK, d), lambda i: (i, 0)),
            pl.BlockSpec((d, npad), lambda i: (0, 0)),
        ],
        out_specs=pl.BlockSpec((_RBLK, n), lambda i: (i, 0)),
        out_shape=jax.ShapeDtypeStruct((n, n), jnp.float32),
        compiler_params=pltpu.CompilerParams(
            dimension_semantics=("arbitrary",)
        ),
    )(nv, nvt)
    return adj
